# baseline (speedup 1.0000x reference)
import functools

import jax
import jax.numpy as jnp
from jax import lax
from jax.experimental import pallas as pl
from jax.experimental.pallas import tpu as pltpu
from jax.experimental.pallas import tpu_sc as plsc

N_CORPUS = 100000
N_PAD = 102400
D_EMB = 128
K_CONCEPTS = 64
KNN_K = 10

CHUNK_N = 12800
GRID_A = N_PAD // CHUNK_N
VREGS_PER_CHUNK = CHUNK_N // 16


def _stage_a_body(te_ref, c_ref, temb_ref, whx_ref, bhx_ref,
                  cross_ref, sqn_ref, yp_ref, l2_ref, nm_ref):
    te = te_ref[...]
    c = c_ref[...]
    cross = lax.dot_general(c, te, (((0,), (1,)), ((), ())),
                            preferred_element_type=jnp.float32)
    cross_ref[...] = cross
    sq = jnp.sum(te * te, axis=1)
    col = lax.broadcasted_iota(jnp.int32, (8, CHUNK_N), 1)
    base = pl.program_id(0) * CHUNK_N
    valid = (base + col) < N_CORPUS
    sqn_ref[...] = jnp.where(valid, jnp.broadcast_to(sq[None, :], (8, CHUNK_N)),
                             jnp.float32(1e30))

    @pl.when(pl.program_id(0) == 0)
    def _dense_head():
        gram = lax.dot_general(c, c, (((0,), (0,)), ((), ())),
                               preferred_element_type=jnp.float32)
        rhs = lax.dot_general(c, whx_ref[...], (((0,), (0,)), ((), ())),
                              preferred_element_type=jnp.float32)
        aug = jnp.concatenate([gram, rhs], axis=1)
        rows = lax.broadcasted_iota(jnp.int32, (K_CONCEPTS, 74), 0)
        for i in range(K_CONCEPTS):
            piv = lax.slice(aug, (i, i), (i + 1, i + 1))
            prow = lax.slice(aug, (i, 0), (i + 1, 74)) / piv
            col = lax.slice(aug, (0, i), (K_CONCEPTS, i + 1))
            aug = jnp.where(rows == i, prow, aug - col * prow)
        z = lax.slice(aug, (0, 64), (K_CONCEPTS, 74))
        m = jnp.dot(c, z, preferred_element_type=jnp.float32)
        yp = jnp.dot(temb_ref[...], m, preferred_element_type=jnp.float32)
        yp_ref[...] = yp + bhx_ref[...]

        r64 = lax.broadcasted_iota(jnp.int32, (K_CONCEPTS, K_CONCEPTS), 0)
        c64 = lax.broadcasted_iota(jnp.int32, (K_CONCEPTS, K_CONCEPTS), 1)
        eye = r64 == c64
        denom = float(K_CONCEPTS * K_CONCEPTS)
        l2_ref[...] = (jnp.sum(jnp.where(eye, 0.0, gram)) / denom).reshape(1, 1)
        nm_ref[...] = (jnp.sum(jnp.where(eye, gram, 0.0)) / denom).reshape(1, 1)


def _run_stage_a(train_embeddings, concept, train_embedding, W_hx, b_hx2d):
    out_shape = [
        jax.ShapeDtypeStruct((K_CONCEPTS, N_PAD), jnp.float32),
        jax.ShapeDtypeStruct((8, N_PAD), jnp.float32),
        jax.ShapeDtypeStruct((train_embedding.shape[0], W_hx.shape[1]),
                             jnp.float32),
        jax.ShapeDtypeStruct((1, 1), jnp.float32),
        jax.ShapeDtypeStruct((1, 1), jnp.float32),
    ]
    zero2 = lambda i: (0, 0)
    return pl.pallas_call(
        _stage_a_body,
        grid=(GRID_A,),
        in_specs=[
            pl.BlockSpec((CHUNK_N, D_EMB), lambda i: (i, 0)),
            pl.BlockSpec((D_EMB, K_CONCEPTS), zero2),
            pl.BlockSpec(train_embedding.shape, zero2),
            pl.BlockSpec(W_hx.shape, zero2),
            pl.BlockSpec(b_hx2d.shape, zero2),
        ],
        out_specs=[
            pl.BlockSpec((K_CONCEPTS, CHUNK_N), lambda i: (0, i)),
            pl.BlockSpec((8, CHUNK_N), lambda i: (0, i)),
            pl.BlockSpec(out_shape[2].shape, zero2),
            pl.BlockSpec((1, 1), zero2),
            pl.BlockSpec((1, 1), zero2),
        ],
        out_shape=out_shape,
    )(train_embeddings, concept, train_embedding, W_hx, b_hx2d)


def _splat_last(v):
    idx = jnp.full((16, 1), 15, jnp.int32)
    dnums = lax.GatherDimensionNumbers(
        offset_dims=(), collapsed_slice_dims=(0,), start_index_map=(0,))
    return lax.gather(v, idx, dnums, slice_sizes=(1,),
                      mode=lax.GatherScatterMode.PROMISE_IN_BOUNDS)


def _merge16(key, val, rk, rv):
    nk, nv = plsc.sort_key_val(key, val, descending=True)
    take = nk < rk
    mk = jnp.where(take, nk, rk)
    mv = jnp.where(take, nv, rv)
    rk2, rv2 = plsc.sort_key_val(mk, mv)
    return rk2, rv2, _splat_last(rk2)


def _row_scan(buf_ref, sqn_ref, state):
    def vstep(j, carry):
        rk, rv, t = carry
        off = pl.multiple_of(j * 16, 16)
        cr = buf_ref[pl.ds(off, 16)]
        sq = sqn_ref[pl.ds(off, 16)]
        key = sq - 2.0 * cr
        hit = jnp.any(key < t)

        def do_merge(_):
            return _merge16(key, cr, rk, rv)

        def skip(_):
            return rk, rv, t

        return lax.cond(hit, do_merge, skip, None)

    return lax.fori_loop(0, VREGS_PER_CHUNK, vstep, state)


def _sc_topk_kernel(cross, sqn):
    mesh = plsc.VectorSubcoreMesh(core_axis_name="c", subcore_axis_name="s")

    @functools.partial(
        pl.kernel,
        mesh=mesh,
        out_type=jax.ShapeDtypeStruct((K_CONCEPTS, 16), jnp.float32),
        scratch_types=[
            pltpu.VMEM((CHUNK_N,), jnp.float32),
            pltpu.VMEM((CHUNK_N,), jnp.float32),
            pltpu.VMEM((CHUNK_N,), jnp.float32),
            pltpu.VMEM((16,), jnp.float32),
        ],
        compiler_params=pltpu.CompilerParams(needs_layout_passes=False),
    )
    def body(cross_hbm, sqn_hbm, out_hbm, c0_v, c1_v, sqn_v, st_v):
        wid = lax.axis_index("s") * 2 + lax.axis_index("c")
        r0 = wid * 2
        r1 = r0 + 1
        inf16 = jnp.full((16,), jnp.inf, jnp.float32)
        init = (inf16, jnp.zeros((16,), jnp.float32), inf16)

        def chunk_body(ci, states):
            s0, s1 = states
            off = pl.multiple_of(ci * CHUNK_N, 16)
            pltpu.sync_copy(sqn_hbm.at[pl.ds(off, CHUNK_N)], sqn_v)
            pltpu.sync_copy(cross_hbm.at[r0, pl.ds(off, CHUNK_N)], c0_v)
            pltpu.sync_copy(cross_hbm.at[r1, pl.ds(off, CHUNK_N)], c1_v)
            s0 = _row_scan(c0_v, sqn_v, s0)
            s1 = _row_scan(c1_v, sqn_v, s1)
            return s0, s1

        s0, s1 = lax.fori_loop(0, N_PAD // CHUNK_N, chunk_body,
                               (init, init))
        keep = lax.iota(jnp.int32, 16) < KNN_K
        st_v[...] = jnp.where(keep, s0[1], 0.0)
        pltpu.sync_copy(st_v, out_hbm.at[r0])
        st_v[...] = jnp.where(keep, s1[1], 0.0)
        pltpu.sync_copy(st_v, out_hbm.at[r1])

    return body(cross, sqn)


def _stage_c_body(d_ref, l1_ref):
    l1_ref[...] = (jnp.sum(d_ref[...]) / float(K_CONCEPTS * KNN_K)).reshape(1, 1)


def _run_stage_c(dots16):
    return pl.pallas_call(
        _stage_c_body,
        out_shape=jax.ShapeDtypeStruct((1, 1), jnp.float32),
    )(dots16)


def kernel(train_embedding, concept, train_embeddings, W_hx, b_hx):
    b_hx2d = b_hx.reshape(1, -1)
    te_pad = jnp.pad(train_embeddings, ((0, N_PAD - N_CORPUS), (0, 0)))
    cross, sqn8, y_pred, l2, nm = _run_stage_a(
        te_pad, concept, train_embedding, W_hx, b_hx2d)
    sqn = sqn8[0]
    dots16 = _sc_topk_kernel(cross, sqn)
    l1 = _run_stage_c(dots16)
    return (y_pred, l1[0, 0], l2[0, 0], nm[0, 0])


# speedup vs baseline: 2.8618x; 2.8618x over previous
import functools

import jax
import jax.numpy as jnp
from jax import lax
from jax.experimental import pallas as pl
from jax.experimental.pallas import tpu as pltpu
from jax.experimental.pallas import tpu_sc as plsc

N_CORPUS = 100000
N_PAD = 102400
D_EMB = 128
K_CONCEPTS = 64
KNN_K = 10

CHUNK_N = 12800
GRID_A = N_PAD // CHUNK_N
VREGS_PER_CHUNK = CHUNK_N // 16


def _stage_a_body(te_ref, c_ref, temb_ref, whx_ref, bhx_ref,
                  cross_ref, sqn_ref, yp_ref, l2_ref, nm_ref):
    te = te_ref[...]
    c = c_ref[...]
    cross = lax.dot_general(c, te, (((0,), (1,)), ((), ())),
                            preferred_element_type=jnp.float32)
    cross_ref[...] = cross
    sq = jnp.sum(te * te, axis=1)
    col = lax.broadcasted_iota(jnp.int32, (8, CHUNK_N), 1)
    base = pl.program_id(0) * CHUNK_N
    valid = (base + col) < N_CORPUS
    sqn_ref[...] = jnp.where(valid, jnp.broadcast_to(sq[None, :], (8, CHUNK_N)),
                             jnp.float32(1e30))

    @pl.when(pl.program_id(0) == 0)
    def _dense_head():
        gram = lax.dot_general(c, c, (((0,), (0,)), ((), ())),
                               preferred_element_type=jnp.float32)
        rhs = lax.dot_general(c, whx_ref[...], (((0,), (0,)), ((), ())),
                              preferred_element_type=jnp.float32)
        aug = jnp.concatenate([gram, rhs], axis=1)
        rows = lax.broadcasted_iota(jnp.int32, (K_CONCEPTS, 74), 0)
        for i in range(K_CONCEPTS):
            piv = lax.slice(aug, (i, i), (i + 1, i + 1))
            prow = lax.slice(aug, (i, 0), (i + 1, 74)) / piv
            col = lax.slice(aug, (0, i), (K_CONCEPTS, i + 1))
            aug = jnp.where(rows == i, prow, aug - col * prow)
        z = lax.slice(aug, (0, 64), (K_CONCEPTS, 74))
        m = jnp.dot(c, z, preferred_element_type=jnp.float32)
        yp = jnp.dot(temb_ref[...], m, preferred_element_type=jnp.float32)
        yp_ref[...] = yp + bhx_ref[...]

        r64 = lax.broadcasted_iota(jnp.int32, (K_CONCEPTS, K_CONCEPTS), 0)
        c64 = lax.broadcasted_iota(jnp.int32, (K_CONCEPTS, K_CONCEPTS), 1)
        eye = r64 == c64
        denom = float(K_CONCEPTS * K_CONCEPTS)
        l2_ref[...] = (jnp.sum(jnp.where(eye, 0.0, gram)) / denom).reshape(1, 1)
        nm_ref[...] = (jnp.sum(jnp.where(eye, gram, 0.0)) / denom).reshape(1, 1)


def _run_stage_a(train_embeddings, concept, train_embedding, W_hx, b_hx2d):
    out_shape = [
        jax.ShapeDtypeStruct((K_CONCEPTS, N_PAD), jnp.float32),
        jax.ShapeDtypeStruct((8, N_PAD), jnp.float32),
        jax.ShapeDtypeStruct((train_embedding.shape[0], W_hx.shape[1]),
                             jnp.float32),
        jax.ShapeDtypeStruct((1, 1), jnp.float32),
        jax.ShapeDtypeStruct((1, 1), jnp.float32),
    ]
    zero2 = lambda i: (0, 0)
    return pl.pallas_call(
        _stage_a_body,
        grid=(GRID_A,),
        in_specs=[
            pl.BlockSpec((CHUNK_N, D_EMB), lambda i: (i, 0)),
            pl.BlockSpec((D_EMB, K_CONCEPTS), zero2),
            pl.BlockSpec(train_embedding.shape, zero2),
            pl.BlockSpec(W_hx.shape, zero2),
            pl.BlockSpec(b_hx2d.shape, zero2),
        ],
        out_specs=[
            pl.BlockSpec((K_CONCEPTS, CHUNK_N), lambda i: (0, i)),
            pl.BlockSpec((8, CHUNK_N), lambda i: (0, i)),
            pl.BlockSpec(out_shape[2].shape, zero2),
            pl.BlockSpec((1, 1), zero2),
            pl.BlockSpec((1, 1), zero2),
        ],
        out_shape=out_shape,
    )(train_embeddings, concept, train_embedding, W_hx, b_hx2d)


def _splat_last(v):
    idx = jnp.full((16, 1), 15, jnp.int32)
    dnums = lax.GatherDimensionNumbers(
        offset_dims=(), collapsed_slice_dims=(0,), start_index_map=(0,))
    return lax.gather(v, idx, dnums, slice_sizes=(1,),
                      mode=lax.GatherScatterMode.PROMISE_IN_BOUNDS)


def _merge16(key, val, rk, rv):
    nk, nv = plsc.sort_key_val(key, val, descending=True)
    take = nk < rk
    mk = jnp.where(take, nk, rk)
    mv = jnp.where(take, nv, rv)
    rk2, rv2 = plsc.sort_key_val(mk, mv)
    return rk2, rv2, _splat_last(rk2)


GROUP = 8
GROUPS_PER_CHUNK = VREGS_PER_CHUNK // GROUP


def _min_tree(vs):
    while len(vs) > 1:
        vs = [jnp.minimum(a, b) for a, b in zip(vs[::2], vs[1::2])] + (
            [vs[-1]] if len(vs) % 2 else [])
    return vs[0]


def _group_update(keys, crs, state):
    rk, rv, t = state
    hit = jnp.any(_min_tree(keys) < t)

    def slow(_):
        s = (rk, rv, t)
        for u in range(GROUP):
            ku, cu = keys[u], crs[u]

            def m(_, s=s, ku=ku, cu=cu):
                return _merge16(ku, cu, s[0], s[1])

            def n(_, s=s):
                return s

            s = lax.cond(jnp.any(ku < s[2]), m, n, None)
        return s

    def fast(_):
        return rk, rv, t

    return lax.cond(hit, slow, fast, None)


def _chunk_scan(c0_ref, c1_ref, sqn_ref, s0, s1):
    def gstep(g, carry):
        st0 = carry[0:3]
        st1 = carry[3:6]
        base = pl.multiple_of(g * (16 * GROUP), 16 * GROUP)
        sqs, k0, k1, c0s, c1s = [], [], [], [], []
        for u in range(GROUP):
            sl = pl.ds(base + u * 16, 16)
            sq = sqn_ref[sl]
            c0 = c0_ref[sl]
            c1 = c1_ref[sl]
            k0.append(sq - c0 - c0)
            k1.append(sq - c1 - c1)
            c0s.append(c0)
            c1s.append(c1)
        st0 = _group_update(k0, c0s, st0)
        st1 = _group_update(k1, c1s, st1)
        return st0 + st1

    out = lax.fori_loop(0, GROUPS_PER_CHUNK, gstep, s0 + s1)
    return out[0:3], out[3:6]


def _sc_topk_kernel(cross, sqn):
    mesh = plsc.VectorSubcoreMesh(core_axis_name="c", subcore_axis_name="s")

    n_chunks = N_PAD // CHUNK_N

    @functools.partial(
        pl.kernel,
        mesh=mesh,
        out_type=jax.ShapeDtypeStruct((K_CONCEPTS, 16), jnp.float32),
        scratch_types=[
            pltpu.VMEM((CHUNK_N,), jnp.float32),
            pltpu.VMEM((CHUNK_N,), jnp.float32),
            pltpu.VMEM((CHUNK_N,), jnp.float32),
            pltpu.VMEM((CHUNK_N,), jnp.float32),
            pltpu.VMEM((CHUNK_N,), jnp.float32),
            pltpu.VMEM((CHUNK_N,), jnp.float32),
            pltpu.VMEM((16,), jnp.float32),
            pltpu.SemaphoreType.DMA,
            pltpu.SemaphoreType.DMA,
        ],
        compiler_params=pltpu.CompilerParams(needs_layout_passes=False),
    )
    def body(cross_hbm, sqn_hbm, out_hbm, sq_a, c0_a, c1_a,
             sq_b, c0_b, c1_b, st_v, sem0, sem1):
        wid = lax.axis_index("s") * 2 + lax.axis_index("c")
        r0 = wid * 2
        r1 = r0 + 1
        bufs = ((sq_a, c0_a, c1_a), (sq_b, c0_b, c1_b))
        sems = (sem0, sem1)
        inf16 = jnp.full((16,), jnp.inf, jnp.float32)
        init = (inf16, jnp.zeros((16,), jnp.float32), inf16)

        def start(ci, slot):
            off = ci * CHUNK_N
            sq_v, c0_v, c1_v = bufs[slot]
            return (
                pltpu.async_copy(sqn_hbm.at[pl.ds(off, CHUNK_N)],
                                 sq_v, sems[slot]),
                pltpu.async_copy(cross_hbm.at[r0, pl.ds(off, CHUNK_N)],
                                 c0_v, sems[slot]),
                pltpu.async_copy(cross_hbm.at[r1, pl.ds(off, CHUNK_N)],
                                 c1_v, sems[slot]),
            )

        s0, s1 = init, init
        handles = [None, None]
        handles[0] = start(0, 0)
        for ci in range(n_chunks):
            slot = ci % 2
            if ci + 1 < n_chunks:
                handles[1 - slot] = start(ci + 1, 1 - slot)
            for h in handles[slot]:
                h.wait()
            sq_v, c0_v, c1_v = bufs[slot]
            s0, s1 = _chunk_scan(c0_v, c1_v, sq_v, s0, s1)

        keep = lax.iota(jnp.int32, 16) < KNN_K
        st_v[...] = jnp.where(keep, s0[1], 0.0)
        pltpu.sync_copy(st_v, out_hbm.at[r0])
        st_v[...] = jnp.where(keep, s1[1], 0.0)
        pltpu.sync_copy(st_v, out_hbm.at[r1])

    return body(cross, sqn)


def _stage_c_body(d_ref, l1_ref):
    l1_ref[...] = (jnp.sum(d_ref[...]) / float(K_CONCEPTS * KNN_K)).reshape(1, 1)


def _run_stage_c(dots16):
    return pl.pallas_call(
        _stage_c_body,
        out_shape=jax.ShapeDtypeStruct((1, 1), jnp.float32),
    )(dots16)


def kernel(train_embedding, concept, train_embeddings, W_hx, b_hx):
    b_hx2d = b_hx.reshape(1, -1)
    te_pad = jnp.pad(train_embeddings, ((0, N_PAD - N_CORPUS), (0, 0)))
    cross, sqn8, y_pred, l2, nm = _run_stage_a(
        te_pad, concept, train_embedding, W_hx, b_hx2d)
    sqn = sqn8[0]
    dots16 = _sc_topk_kernel(cross, sqn)
    l1 = _run_stage_c(dots16)
    return (y_pred, l1[0, 0], l2[0, 0], nm[0, 0])


# speedup vs baseline: 3.2607x; 1.1394x over previous
import functools

import jax
import jax.numpy as jnp
from jax import lax
from jax.experimental import pallas as pl
from jax.experimental.pallas import tpu as pltpu
from jax.experimental.pallas import tpu_sc as plsc

N_CORPUS = 100000
N_PAD = 102400
D_EMB = 128
K_CONCEPTS = 64
KNN_K = 10

CHUNK_N = 12800
GRID_A = N_PAD // CHUNK_N
VREGS_PER_CHUNK = CHUNK_N // 16


def _stage_a_body(te_ref, c_ref, temb_ref, whx_ref, bhx_ref,
                  cross_ref, sqn_ref, yp_ref, l2_ref, nm_ref):
    te = te_ref[...]
    c = c_ref[...]
    cross = lax.dot_general(c, te, (((0,), (1,)), ((), ())),
                            preferred_element_type=jnp.float32)
    base = pl.program_id(0) * CHUNK_N
    colk = lax.broadcasted_iota(jnp.int32, (K_CONCEPTS, CHUNK_N), 1)
    cross_ref[...] = jnp.where(base + colk < N_CORPUS, cross, 0.0)
    sq = jnp.sum(te * te, axis=1)
    col1 = lax.broadcasted_iota(jnp.int32, (1, CHUNK_N), 1)
    sqn_ref[...] = jnp.where(base + col1 < N_CORPUS, sq[None, :],
                             jnp.float32(1e30))

    @pl.when(pl.program_id(0) == 0)
    def _dense_head():
        gram = lax.dot_general(c, c, (((0,), (0,)), ((), ())),
                               preferred_element_type=jnp.float32)
        rhs = lax.dot_general(c, whx_ref[...], (((0,), (0,)), ((), ())),
                              preferred_element_type=jnp.float32)
        aug = jnp.concatenate([gram, rhs], axis=1)
        rows = lax.broadcasted_iota(jnp.int32, (K_CONCEPTS, 74), 0)
        for i in range(K_CONCEPTS):
            piv = lax.slice(aug, (i, i), (i + 1, i + 1))
            prow = lax.slice(aug, (i, 0), (i + 1, 74)) / piv
            col = lax.slice(aug, (0, i), (K_CONCEPTS, i + 1))
            aug = jnp.where(rows == i, prow, aug - col * prow)
        z = lax.slice(aug, (0, 64), (K_CONCEPTS, 74))
        m = jnp.dot(c, z, preferred_element_type=jnp.float32)
        yp = jnp.dot(temb_ref[...], m, preferred_element_type=jnp.float32)
        yp_ref[...] = yp + bhx_ref[...]

        r64 = lax.broadcasted_iota(jnp.int32, (K_CONCEPTS, K_CONCEPTS), 0)
        c64 = lax.broadcasted_iota(jnp.int32, (K_CONCEPTS, K_CONCEPTS), 1)
        eye = r64 == c64
        denom = float(K_CONCEPTS * K_CONCEPTS)
        l2_ref[...] = (jnp.sum(jnp.where(eye, 0.0, gram)) / denom).reshape(1, 1)
        nm_ref[...] = (jnp.sum(jnp.where(eye, gram, 0.0)) / denom).reshape(1, 1)


def _run_stage_a(train_embeddings, concept, train_embedding, W_hx, b_hx2d):
    out_shape = [
        jax.ShapeDtypeStruct((K_CONCEPTS, N_PAD), jnp.float32),
        jax.ShapeDtypeStruct((1, N_PAD), jnp.float32),
        jax.ShapeDtypeStruct((train_embedding.shape[0], W_hx.shape[1]),
                             jnp.float32),
        jax.ShapeDtypeStruct((1, 1), jnp.float32),
        jax.ShapeDtypeStruct((1, 1), jnp.float32),
    ]
    zero2 = lambda i: (0, 0)
    return pl.pallas_call(
        _stage_a_body,
        grid=(GRID_A,),
        in_specs=[
            pl.BlockSpec((CHUNK_N, D_EMB), lambda i: (i, 0)),
            pl.BlockSpec((D_EMB, K_CONCEPTS), zero2),
            pl.BlockSpec(train_embedding.shape, zero2),
            pl.BlockSpec(W_hx.shape, zero2),
            pl.BlockSpec(b_hx2d.shape, zero2),
        ],
        out_specs=[
            pl.BlockSpec((K_CONCEPTS, CHUNK_N), lambda i: (0, i)),
            pl.BlockSpec((1, CHUNK_N), lambda i: (0, i)),
            pl.BlockSpec(out_shape[2].shape, zero2),
            pl.BlockSpec((1, 1), zero2),
            pl.BlockSpec((1, 1), zero2),
        ],
        out_shape=out_shape,
    )(train_embeddings, concept, train_embedding, W_hx, b_hx2d)


def _splat_last(v):
    idx = jnp.full((16, 1), 15, jnp.int32)
    dnums = lax.GatherDimensionNumbers(
        offset_dims=(), collapsed_slice_dims=(0,), start_index_map=(0,))
    return lax.gather(v, idx, dnums, slice_sizes=(1,),
                      mode=lax.GatherScatterMode.PROMISE_IN_BOUNDS)


def _merge16(key, val, rk, rv):
    nk, nv = plsc.sort_key_val(key, val, descending=True)
    take = nk < rk
    mk = jnp.where(take, nk, rk)
    mv = jnp.where(take, nv, rv)
    rk2, rv2 = plsc.sort_key_val(mk, mv)
    return rk2, rv2, _splat_last(rk2)


GROUP = 16
GROUPS_PER_CHUNK = VREGS_PER_CHUNK // GROUP


def _min_tree(vs):
    while len(vs) > 1:
        vs = [jnp.minimum(a, b) for a, b in zip(vs[::2], vs[1::2])] + (
            [vs[-1]] if len(vs) % 2 else [])
    return vs[0]


def _any16(mask):
    cnt = plsc.all_reduce_population_count(mask)
    return (cnt[0] if cnt.ndim else cnt) > 0


def _group_update(keys, crs, state):
    rk, rv, t = state
    hit = _any16(_min_tree(keys) < t)

    def slow(_):
        s = (rk, rv, t)
        for u in range(GROUP):
            ku, cu = keys[u], crs[u]

            def m(_, s=s, ku=ku, cu=cu):
                return _merge16(ku, cu, s[0], s[1])

            def n(_, s=s):
                return s

            s = lax.cond(jnp.any(ku < s[2]), m, n, None)
        return s

    def fast(_):
        return rk, rv, t

    return lax.cond(hit, slow, fast, None)


def _chunk_scan(c0_ref, c1_ref, sqn_ref, s0, s1):
    def gstep(g, carry):
        st0 = carry[0:3]
        st1 = carry[3:6]
        base = pl.multiple_of(g * (16 * GROUP), 16 * GROUP)
        sqs, k0, k1, c0s, c1s = [], [], [], [], []
        for u in range(GROUP):
            sl = pl.ds(base + u * 16, 16)
            sq = sqn_ref[sl]
            c0 = c0_ref[sl]
            c1 = c1_ref[sl]
            k0.append(sq - c0 - c0)
            k1.append(sq - c1 - c1)
            c0s.append(c0)
            c1s.append(c1)
        st0 = _group_update(k0, c0s, st0)
        st1 = _group_update(k1, c1s, st1)
        return st0 + st1

    out = lax.fori_loop(0, GROUPS_PER_CHUNK, gstep, s0 + s1)
    return out[0:3], out[3:6]


def _sc_topk_kernel(cross, sqn):
    mesh = plsc.VectorSubcoreMesh(core_axis_name="c", subcore_axis_name="s")

    n_chunks = N_PAD // CHUNK_N

    @functools.partial(
        pl.kernel,
        mesh=mesh,
        out_type=jax.ShapeDtypeStruct((K_CONCEPTS, 16), jnp.float32),
        scratch_types=[
            pltpu.VMEM((CHUNK_N,), jnp.float32),
            pltpu.VMEM((CHUNK_N,), jnp.float32),
            pltpu.VMEM((CHUNK_N,), jnp.float32),
            pltpu.VMEM((CHUNK_N,), jnp.float32),
            pltpu.VMEM((CHUNK_N,), jnp.float32),
            pltpu.VMEM((CHUNK_N,), jnp.float32),
            pltpu.VMEM((16,), jnp.float32),
            pltpu.SemaphoreType.DMA,
            pltpu.SemaphoreType.DMA,
        ],
        compiler_params=pltpu.CompilerParams(needs_layout_passes=False),
    )
    def body(cross_hbm, sqn_hbm, out_hbm, sq_a, c0_a, c1_a,
             sq_b, c0_b, c1_b, st_v, sem0, sem1):
        wid = lax.axis_index("s") * 2 + lax.axis_index("c")
        r0 = wid * 2
        r1 = r0 + 1
        bufs = ((sq_a, c0_a, c1_a), (sq_b, c0_b, c1_b))
        sems = (sem0, sem1)
        inf16 = jnp.full((16,), jnp.inf, jnp.float32)
        init = (inf16, jnp.zeros((16,), jnp.float32), inf16)

        def start(ci, slot):
            off = ci * CHUNK_N
            sq_v, c0_v, c1_v = bufs[slot]
            return (
                pltpu.async_copy(sqn_hbm.at[pl.ds(off, CHUNK_N)],
                                 sq_v, sems[slot]),
                pltpu.async_copy(cross_hbm.at[r0, pl.ds(off, CHUNK_N)],
                                 c0_v, sems[slot]),
                pltpu.async_copy(cross_hbm.at[r1, pl.ds(off, CHUNK_N)],
                                 c1_v, sems[slot]),
            )

        s0, s1 = init, init
        handles = [None, None]
        handles[0] = start(0, 0)
        for ci in range(n_chunks):
            slot = ci % 2
            if ci + 1 < n_chunks:
                handles[1 - slot] = start(ci + 1, 1 - slot)
            for h in handles[slot]:
                h.wait()
            sq_v, c0_v, c1_v = bufs[slot]
            s0, s1 = _chunk_scan(c0_v, c1_v, sq_v, s0, s1)

        keep = lax.iota(jnp.int32, 16) < KNN_K
        st_v[...] = jnp.where(keep, s0[1], 0.0)
        pltpu.sync_copy(st_v, out_hbm.at[r0])
        st_v[...] = jnp.where(keep, s1[1], 0.0)
        pltpu.sync_copy(st_v, out_hbm.at[r1])

    return body(cross, sqn)


def _stage_c_body(d_ref, l1_ref):
    l1_ref[...] = (jnp.sum(d_ref[...]) / float(K_CONCEPTS * KNN_K)).reshape(1, 1)


def _run_stage_c(dots16):
    return pl.pallas_call(
        _stage_c_body,
        out_shape=jax.ShapeDtypeStruct((1, 1), jnp.float32),
    )(dots16)


def kernel(train_embedding, concept, train_embeddings, W_hx, b_hx):
    b_hx2d = b_hx.reshape(1, -1)
    cross, sqn1, y_pred, l2, nm = _run_stage_a(
        train_embeddings, concept, train_embedding, W_hx, b_hx2d)
    sqn = sqn1.reshape(N_PAD)
    dots16 = _sc_topk_kernel(cross, sqn)
    l1 = _run_stage_c(dots16)
    return (y_pred, l1[0, 0], l2[0, 0], nm[0, 0])


# speedup vs baseline: 3.5772x; 1.0971x over previous
import functools

import jax
import jax.numpy as jnp
from jax import lax
from jax.experimental import pallas as pl
from jax.experimental.pallas import tpu as pltpu
from jax.experimental.pallas import tpu_sc as plsc

N_CORPUS = 100000
N_PAD = 102400
D_EMB = 128
K_CONCEPTS = 64
KNN_K = 10

CHUNK_N = 12800
GRID_A = N_PAD // CHUNK_N
VREGS_PER_CHUNK = CHUNK_N // 16


def _stage_a_body(te_ref, c_ref, temb_ref, whx_ref, bhx_ref,
                  keys_ref, yp_ref, l2_ref, nm_ref):
    te = te_ref[...]
    c = c_ref[...]
    cross = lax.dot_general(c, te, (((0,), (1,)), ((), ())),
                            preferred_element_type=jnp.float32)
    ones1 = jnp.ones((1, D_EMB), jnp.float32)
    sqrow = lax.dot_general(ones1, te * te, (((1,), (1,)), ((), ())),
                            preferred_element_type=jnp.float32)
    keys = sqrow - 2.0 * cross
    base = pl.program_id(0) * CHUNK_N
    colk = lax.broadcasted_iota(jnp.int32, (K_CONCEPTS, CHUNK_N), 1)
    keys_ref[...] = jnp.where(base + colk < N_CORPUS, keys, jnp.float32(1e30))

    @pl.when(pl.program_id(0) == 0)
    def _dense_head():
        gram = lax.dot_general(c, c, (((0,), (0,)), ((), ())),
                               preferred_element_type=jnp.float32)
        rhs = lax.dot_general(c, whx_ref[...], (((0,), (0,)), ((), ())),
                              preferred_element_type=jnp.float32)
        aug = jnp.concatenate([gram, rhs], axis=1)
        rows = lax.broadcasted_iota(jnp.int32, (K_CONCEPTS, 74), 0)
        for i in range(K_CONCEPTS):
            piv = lax.slice(aug, (i, i), (i + 1, i + 1))
            prow = lax.slice(aug, (i, 0), (i + 1, 74)) / piv
            col = lax.slice(aug, (0, i), (K_CONCEPTS, i + 1))
            aug = jnp.where(rows == i, prow, aug - col * prow)
        z = lax.slice(aug, (0, 64), (K_CONCEPTS, 74))
        m = jnp.dot(c, z, preferred_element_type=jnp.float32)
        yp = jnp.dot(temb_ref[...], m, preferred_element_type=jnp.float32)
        yp_ref[...] = yp + bhx_ref[...]

        r64 = lax.broadcasted_iota(jnp.int32, (K_CONCEPTS, K_CONCEPTS), 0)
        c64 = lax.broadcasted_iota(jnp.int32, (K_CONCEPTS, K_CONCEPTS), 1)
        eye = r64 == c64
        denom = float(K_CONCEPTS * K_CONCEPTS)
        l2_ref[...] = (jnp.sum(jnp.where(eye, 0.0, gram)) / denom).reshape(1, 1)
        nm_ref[...] = (jnp.sum(jnp.where(eye, gram, 0.0)) / denom).reshape(1, 1)


def _run_stage_a(train_embeddings, concept, train_embedding, W_hx, b_hx2d):
    out_shape = [
        jax.ShapeDtypeStruct((K_CONCEPTS, N_PAD), jnp.float32),
        jax.ShapeDtypeStruct((train_embedding.shape[0], W_hx.shape[1]),
                             jnp.float32),
        jax.ShapeDtypeStruct((1, 1), jnp.float32),
        jax.ShapeDtypeStruct((1, 1), jnp.float32),
    ]
    zero2 = lambda i: (0, 0)
    return pl.pallas_call(
        _stage_a_body,
        grid=(GRID_A,),
        in_specs=[
            pl.BlockSpec((CHUNK_N, D_EMB), lambda i: (i, 0)),
            pl.BlockSpec((D_EMB, K_CONCEPTS), zero2),
            pl.BlockSpec(train_embedding.shape, zero2),
            pl.BlockSpec(W_hx.shape, zero2),
            pl.BlockSpec(b_hx2d.shape, zero2),
        ],
        out_specs=[
            pl.BlockSpec((K_CONCEPTS, CHUNK_N), lambda i: (0, i)),
            pl.BlockSpec(out_shape[1].shape, zero2),
            pl.BlockSpec((1, 1), zero2),
            pl.BlockSpec((1, 1), zero2),
        ],
        out_shape=out_shape,
    )(train_embeddings, concept, train_embedding, W_hx, b_hx2d)


def _splat_last(v):
    idx = jnp.full((16, 1), 15, jnp.int32)
    dnums = lax.GatherDimensionNumbers(
        offset_dims=(), collapsed_slice_dims=(0,), start_index_map=(0,))
    return lax.gather(v, idx, dnums, slice_sizes=(1,),
                      mode=lax.GatherScatterMode.PROMISE_IN_BOUNDS)


def _merge16(key, val, rk, rv):
    nk, nv = plsc.sort_key_val(key, val, descending=True)
    take = nk < rk
    mk = jnp.where(take, nk, rk)
    mv = jnp.where(take, nv, rv)
    rk2, rv2 = plsc.sort_key_val(mk, mv)
    return rk2, rv2, _splat_last(rk2)


GROUP = 16
GROUPS_PER_CHUNK = VREGS_PER_CHUNK // GROUP


def _min_tree(vs):
    while len(vs) > 1:
        vs = [jnp.minimum(a, b) for a, b in zip(vs[::2], vs[1::2])] + (
            [vs[-1]] if len(vs) % 2 else [])
    return vs[0]


def _any16(mask):
    cnt = plsc.all_reduce_population_count(mask)
    return (cnt[0] if cnt.ndim else cnt) > 0


def _group_update(keys, base_n, state):
    rk, ri, t = state
    hit = _any16(_min_tree(keys) < t)
    lane = lax.iota(jnp.int32, 16)

    def slow(_):
        s = (rk, ri, t)
        for u in range(GROUP):
            ku = keys[u]
            iu = base_n + (u * 16) + lane

            def m(_, s=s, ku=ku, iu=iu):
                return _merge16(ku, iu, s[0], s[1])

            def n(_, s=s):
                return s

            s = lax.cond(_any16(ku < s[2]), m, n, None)
        return s

    def fast(_):
        return rk, ri, t

    return lax.cond(hit, slow, fast, None)


def _chunk_scan(k0_ref, k1_ref, chunk_off, s0, s1):
    def gstep(g, carry):
        st0 = carry[0:3]
        st1 = carry[3:6]
        base = pl.multiple_of(g * (16 * GROUP), 16 * GROUP)
        k0 = [k0_ref[pl.ds(base + u * 16, 16)] for u in range(GROUP)]
        k1 = [k1_ref[pl.ds(base + u * 16, 16)] for u in range(GROUP)]
        base_n = chunk_off + base
        st0 = _group_update(k0, base_n, st0)
        st1 = _group_update(k1, base_n, st1)
        return st0 + st1

    out = lax.fori_loop(0, GROUPS_PER_CHUNK, gstep, s0 + s1)
    return out[0:3], out[3:6]


def _sc_topk_kernel(keys, concT, te):
    mesh = plsc.VectorSubcoreMesh(core_axis_name="c", subcore_axis_name="s")

    n_chunks = N_PAD // CHUNK_N

    @functools.partial(
        pl.kernel,
        mesh=mesh,
        out_type=jax.ShapeDtypeStruct((K_CONCEPTS, 16), jnp.float32),
        scratch_types=[
            pltpu.VMEM((CHUNK_N,), jnp.float32),
            pltpu.VMEM((CHUNK_N,), jnp.float32),
            pltpu.VMEM((CHUNK_N,), jnp.float32),
            pltpu.VMEM((CHUNK_N,), jnp.float32),
            pltpu.VMEM((D_EMB,), jnp.float32),
            pltpu.VMEM((16,), jnp.int32),
            pltpu.VMEM((16, D_EMB), jnp.float32),
            pltpu.VMEM((16,), jnp.float32),
            pltpu.SemaphoreType.DMA,
            pltpu.SemaphoreType.DMA,
            pltpu.SemaphoreType.DMA,
        ],
        compiler_params=pltpu.CompilerParams(needs_layout_passes=False),
    )
    def body(keys_hbm, concT_hbm, te_hbm, out_hbm, k0_a, k1_a,
             k0_b, k1_b, conc_v, idx_v, rows_v, st_v, sem0, sem1, semg):
        wid = lax.axis_index("s") * 2 + lax.axis_index("c")
        r0 = wid * 2
        r1 = r0 + 1
        bufs = ((k0_a, k1_a), (k0_b, k1_b))
        sems = (sem0, sem1)
        inf16 = jnp.full((16,), jnp.inf, jnp.float32)
        init = (inf16, jnp.zeros((16,), jnp.int32), inf16)

        def start(ci, slot):
            off = ci * CHUNK_N
            k0_v, k1_v = bufs[slot]
            return (
                pltpu.async_copy(keys_hbm.at[r0, pl.ds(off, CHUNK_N)],
                                 k0_v, sems[slot]),
                pltpu.async_copy(keys_hbm.at[r1, pl.ds(off, CHUNK_N)],
                                 k1_v, sems[slot]),
            )

        s0, s1 = init, init
        handles = [None, None]
        handles[0] = start(0, 0)
        for ci in range(n_chunks):
            slot = ci % 2
            if ci + 1 < n_chunks:
                handles[1 - slot] = start(ci + 1, 1 - slot)
            for h in handles[slot]:
                h.wait()
            k0_v, k1_v = bufs[slot]
            s0, s1 = _chunk_scan(k0_v, k1_v, ci * CHUNK_N, s0, s1)

        lane = lax.iota(jnp.int32, 16)

        def finish(state, row):
            idx_v[...] = state[1]
            pltpu.sync_copy(concT_hbm.at[row], conc_v)
            pltpu.async_copy(te_hbm.at[idx_v], rows_v, semg).wait()
            acc = jnp.zeros((16,), jnp.float32)
            for d8 in range(D_EMB // 16):
                cs = conc_v[pl.ds(d8 * 16, 16)]
                for j in range(KNN_K):
                    acc = acc + rows_v[j, pl.ds(d8 * 16, 16)] * cs
            total = jnp.sum(acc)
            st_v[...] = jnp.where(lane == 0, total, 0.0)
            pltpu.sync_copy(st_v, out_hbm.at[row])

        finish(s0, r0)
        finish(s1, r1)

    return body(keys, concT, te)


def _stage_c_body(d_ref, l1_ref):
    l1_ref[...] = (jnp.sum(d_ref[...]) / float(K_CONCEPTS * KNN_K)).reshape(1, 1)


def _run_stage_c(dots16):
    return pl.pallas_call(
        _stage_c_body,
        out_shape=jax.ShapeDtypeStruct((1, 1), jnp.float32),
    )(dots16)


def kernel(train_embedding, concept, train_embeddings, W_hx, b_hx):
    b_hx2d = b_hx.reshape(1, -1)
    keys, y_pred, l2, nm = _run_stage_a(
        train_embeddings, concept, train_embedding, W_hx, b_hx2d)
    concT = concept.T
    dots16 = _sc_topk_kernel(keys, concT, train_embeddings)
    l1 = _run_stage_c(dots16)
    return (y_pred, l1[0, 0], l2[0, 0], nm[0, 0])
